# baseline (speedup 1.0000x reference)
import functools

import jax
import jax.numpy as jnp
from jax import lax
from jax.experimental import pallas as pl
from jax.experimental.pallas import tpu as pltpu
from jax.experimental.pallas import tpu_sc as plsc

_D = 64
_B = 4096
_NC = 2
_NS = 16
_NW = _NC * _NS
_C = 4
_V = 1000000
_T0 = (_V // 128) * 128
_TW = _V - _T0

_mesh = plsc.VectorSubcoreMesh(core_axis_name="c", subcore_axis_name="s")


def _make_gather(n, bmajor=False):
    npw = n // _NW
    ng = npw // 16
    out_t = (jax.ShapeDtypeStruct((n, _D), jnp.float32) if bmajor
             else jax.ShapeDtypeStruct((8, _D // 8, n), jnp.float32))

    @functools.partial(
        pl.kernel,
        out_type=out_t,
        mesh=_mesh,
        scratch_types=[
            pltpu.VMEM((npw,), jnp.int32),
            pltpu.VMEM((8, _D // 8, 3 * _C * 128), jnp.float32),
            (pltpu.VMEM((npw, _D), jnp.float32) if bmajor
             else pltpu.VMEM((8, _D // 8, npw), jnp.float32)),
            pltpu.SemaphoreType.DMA,
        ],
        compiler_params=pltpu.CompilerParams(needs_layout_passes=False),
    )
    def gather(idx_hbm, t3, tail3, out, idx_v, buf, rows_v, sem):
        wid = lax.axis_index("s") * _NC + lax.axis_index("c")
        base = wid * npw
        pltpu.sync_copy(idx_hbm.at[pl.ds(base, npw)], idx_v)

        lane = lax.iota(jnp.int32, 16)
        avec = [(lane + q * 16) >> 3 for q in range(4)]
        svec = [(lane + q * 16) & 7 for q in range(4)]

        def fetch_one(idx, slot):
            tail = idx >= _T0
            v0 = pl.multiple_of(idx & -128, 128)

            @pl.when(jnp.logical_not(tail))
            def _():
                pltpu.async_copy(t3.at[:, :, pl.ds(v0, 128)],
                                 buf.at[:, :, pl.ds(slot * 128, 128)], sem)

            @pl.when(tail)
            def _():
                pltpu.async_copy(tail3,
                                 buf.at[:, :, pl.ds(slot * 128, 128)], sem)

        def wait_one(slot):
            pltpu.make_async_copy(
                t3.at[:, :, pl.ds(0, 128)],
                buf.at[:, :, pl.ds(slot * 128, 128)], sem).wait()

        def gather_group(g, _):
            vec = idx_v[pl.ds(g * 16, 16)]
            nch = 16 // _C

            def fire(c):
                for j in range(_C):
                    fetch_one(vec[c * _C + j], (c % 3) * _C + j)

            def drain_extract(c):
                for j in range(_C):
                    wait_one((c % 3) * _C + j)
                for j in range(_C):
                    slot = (c % 3) * _C + j
                    x = vec[c * _C + j]
                    b = g * 16 + c * _C + j
                    v0 = jnp.where(x >= _T0, _T0, x & -128)
                    pos = jnp.full((16,), slot * 128, jnp.int32) + (x - v0)
                    bvec = jnp.full((16,), b, jnp.int32)
                    for q in range(4):
                        vals = plsc.load_gather(buf, [avec[q], svec[q], pos])
                        if bmajor:
                            plsc.store_scatter(rows_v, [bvec, lane + q * 16], vals)
                        else:
                            plsc.store_scatter(rows_v, [avec[q], svec[q], bvec], vals)

            fire(0)
            fire(1)
            for c in range(2, nch):
                fire(c)
                drain_extract(c - 2)
            drain_extract(nch - 2)
            drain_extract(nch - 1)
            return 0

        lax.fori_loop(0, ng, gather_group, 0)
        if bmajor:
            pltpu.sync_copy(rows_v, out.at[pl.ds(base, npw)])
        else:
            pltpu.sync_copy(rows_v, out.at[:, :, pl.ds(base, npw)])

    return gather


_gather_full = _make_gather(_B)
_gather_half = _make_gather(_B // 2, bmajor=True)

_BN = 512
_BM = _B // 2


def _mm_body(wu_ref, hi_ref, out_ref):
    out_ref[...] = lax.dot_general(
        wu_ref[...], hi_ref[...],
        (((1,), (0,)), ((), ())),
        preferred_element_type=jnp.float32,
    )


def _mm_body_bot(wu_ref, hi_ref, prev_ref, out_ref):
    del prev_ref
    out_ref[...] = lax.dot_general(
        wu_ref[...], hi_ref[...],
        (((1,), (0,)), ((), ())),
        preferred_element_type=jnp.float32,
    )


_matmul_top = pl.pallas_call(
    _mm_body,
    grid=(_B // _BN,),
    in_specs=[
        pl.BlockSpec((_BM, _D), lambda j: (0, 0)),
        pl.BlockSpec((_D, _BN), lambda j: (0, j)),
    ],
    out_specs=pl.BlockSpec((_BM, _BN), lambda j: (0, j)),
    out_shape=jax.ShapeDtypeStruct((_B, _B), jnp.float32),
)

_matmul_bot = pl.pallas_call(
    _mm_body_bot,
    grid=(_B // _BN,),
    in_specs=[
        pl.BlockSpec((_BM, _D), lambda j: (0, 0)),
        pl.BlockSpec((_D, _BN), lambda j: (0, j)),
        pl.BlockSpec(memory_space=pltpu.MemorySpace.HBM),
    ],
    out_specs=pl.BlockSpec((_BM, _BN), lambda j: (1, j)),
    out_shape=jax.ShapeDtypeStruct((_B, _B), jnp.float32),
    input_output_aliases={2: 0},
)


@jax.jit
def kernel(user_tensor, item_tensor, user_embedding, item_embedding):
    uidx = user_tensor.astype(jnp.int32)
    iidx = item_tensor.astype(jnp.int32)
    nu = user_embedding.shape[0]
    ni = item_embedding.shape[0]
    u3 = user_embedding.T.reshape(8, _D // 8, nu)
    i3 = item_embedding.T.reshape(8, _D // 8, ni)
    ut3 = jnp.pad(u3[:, :, _T0:], ((0, 0), (0, 0), (0, 128 - _TW)))
    it3 = jnp.pad(i3[:, :, _T0:], ((0, 0), (0, 0), (0, 128 - _TW)))
    hi3 = _gather_full(iidx, i3, it3)
    wu3a = _gather_half(uidx[:_BM], u3, ut3)
    wu3b = _gather_half(uidx[_BM:], u3, ut3)
    hi_t = hi3.reshape(_D, _B)
    o1 = _matmul_top(wu3a, hi_t)
    return _matmul_bot(wu3b, hi_t, o1)


# speedup vs baseline: 1.0908x; 1.0908x over previous
import functools

import jax
import jax.numpy as jnp
from jax import lax
from jax.experimental import pallas as pl
from jax.experimental.pallas import tpu as pltpu
from jax.experimental.pallas import tpu_sc as plsc

_D = 64
_B = 4096
_NC = 2
_NS = 16
_NW = _NC * _NS
_BPW = _B // _NW
_G = _BPW // 16
_C = 2
_V = 1000000
_T0 = (_V // 128) * 128
_TW = _V - _T0

_mesh = plsc.VectorSubcoreMesh(core_axis_name="c", subcore_axis_name="s")

@functools.partial(
    pl.kernel,
    out_type=(
        jax.ShapeDtypeStruct((8, _D // 8, _B), jnp.float32),
        jax.ShapeDtypeStruct((8, _D // 8, _B), jnp.float32),
    ),
    mesh=_mesh,
    scratch_types=[
        pltpu.VMEM((_BPW,), jnp.int32),
        pltpu.VMEM((_BPW,), jnp.int32),
        pltpu.VMEM((8, _D // 8, 3 * _C * 128), jnp.float32),
        pltpu.VMEM((8, _D // 8, 3 * _C * 128), jnp.float32),
        pltpu.VMEM((8, _D // 8, _BPW), jnp.float32),
        pltpu.VMEM((8, _D // 8, _BPW), jnp.float32),
        pltpu.SemaphoreType.DMA,
        pltpu.SemaphoreType.DMA,
    ],
    compiler_params=pltpu.CompilerParams(needs_layout_passes=False),
)
def _sc_gather(user_idx_hbm, item_idx_hbm, u3, i3, ut3, it3,
               wu_out, hi_out, uidx_v, iidx_v, ubuf, ibuf,
               urows_v, irows_v, usem, isem):
    wid = lax.axis_index("s") * _NC + lax.axis_index("c")
    base = wid * _BPW
    pltpu.sync_copy(user_idx_hbm.at[pl.ds(base, _BPW)], uidx_v)
    pltpu.sync_copy(item_idx_hbm.at[pl.ds(base, _BPW)], iidx_v)

    def fetch_one(tab, tailtab, buf, sem, idx, j):
        tail = idx >= _T0
        v0 = pl.multiple_of(idx & -128, 128)

        @pl.when(jnp.logical_not(tail))
        def _():
            pltpu.async_copy(tab.at[:, :, pl.ds(v0, 128)],
                             buf.at[:, :, pl.ds(j * 128, 128)], sem)

        @pl.when(tail)
        def _():
            pltpu.async_copy(tailtab,
                             buf.at[:, :, pl.ds(j * 128, 128)], sem)

    def wait_one(tab, buf, sem, j):
        pltpu.make_async_copy(tab.at[:, :, pl.ds(0, 128)],
                              buf.at[:, :, pl.ds(j * 128, 128)], sem).wait()

    lane = lax.iota(jnp.int32, 16)
    avec = [(lane + q * 16) >> 3 for q in range(4)]
    svec = [(lane + q * 16) & 7 for q in range(4)]

    def gather_group(g, _):
        uvec = uidx_v[pl.ds(g * 16, 16)]
        ivec = iidx_v[pl.ds(g * 16, 16)]
        nch = 16 // _C

        def fire(c):
            for j in range(_C):
                slot = (c % 3) * _C + j
                fetch_one(u3, ut3, ubuf, usem, uvec[c * _C + j], slot)
                fetch_one(i3, it3, ibuf, isem, ivec[c * _C + j], slot)

        def drain_extract(c):
            for j in range(_C):
                slot = (c % 3) * _C + j
                wait_one(u3, ubuf, usem, slot)
                wait_one(i3, ibuf, isem, slot)
            for j in range(_C):
                slot = (c % 3) * _C + j
                uj = uvec[c * _C + j]
                ij = ivec[c * _C + j]
                b = g * 16 + c * _C + j
                uv0 = jnp.where(uj >= _T0, _T0, uj & -128)
                iv0 = jnp.where(ij >= _T0, _T0, ij & -128)
                upos = jnp.full((16,), slot * 128, jnp.int32) + (uj - uv0)
                ipos = jnp.full((16,), slot * 128, jnp.int32) + (ij - iv0)
                bvec = jnp.full((16,), b, jnp.int32)
                for q in range(4):
                    uvals = plsc.load_gather(ubuf, [avec[q], svec[q], upos])
                    plsc.store_scatter(urows_v, [avec[q], svec[q], bvec], uvals)
                    ivals = plsc.load_gather(ibuf, [avec[q], svec[q], ipos])
                    plsc.store_scatter(irows_v, [avec[q], svec[q], bvec], ivals)

        fire(0)
        fire(1)
        for c in range(2, nch):
            fire(c)
            drain_extract(c - 2)
        drain_extract(nch - 2)
        drain_extract(nch - 1)
        return 0

    lax.fori_loop(0, _G, gather_group, 0)
    pltpu.sync_copy(urows_v, wu_out.at[:, :, pl.ds(base, _BPW)])
    pltpu.sync_copy(irows_v, hi_out.at[:, :, pl.ds(base, _BPW)])


_BN = 512


def _mm_body(wu_ref, hi_ref, out_ref):
    out_ref[...] = lax.dot_general(
        wu_ref[...], hi_ref[...],
        (((0,), (0,)), ((), ())),
        preferred_element_type=jnp.float32,
    )


_matmul = pl.pallas_call(
    _mm_body,
    grid=(_B // _BN,),
    in_specs=[
        pl.BlockSpec((_D, _B), lambda j: (0, 0)),
        pl.BlockSpec((_D, _BN), lambda j: (0, j)),
    ],
    out_specs=pl.BlockSpec((_B, _BN), lambda j: (0, j)),
    out_shape=jax.ShapeDtypeStruct((_B, _B), jnp.float32),
)


@jax.jit
def kernel(user_tensor, item_tensor, user_embedding, item_embedding):
    uidx = user_tensor.astype(jnp.int32)
    iidx = item_tensor.astype(jnp.int32)
    nu = user_embedding.shape[0]
    ni = item_embedding.shape[0]
    u3 = user_embedding.T.reshape(8, _D // 8, nu)
    i3 = item_embedding.T.reshape(8, _D // 8, ni)
    ut3 = jnp.pad(u3[:, :, _T0:], ((0, 0), (0, 0), (0, 128 - _TW)))
    it3 = jnp.pad(i3[:, :, _T0:], ((0, 0), (0, 0), (0, 128 - _TW)))
    wu3, hi3 = _sc_gather(uidx, iidx, u3, i3, ut3, it3)
    return _matmul(wu3.reshape(_D, _B), hi3.reshape(_D, _B))
